# Initial kernel scaffold; baseline (speedup 1.0000x reference)
import jax
import jax.numpy as jnp
from jax.experimental import pallas as pl


def kernel(d_vals, sdf):
    raise NotImplementedError("write your pallas kernel here")


# speedup vs baseline: 2.8927x; 2.8927x over previous
import functools

import jax
import jax.numpy as jnp
from jax import lax
from jax.experimental import pallas as pl
from jax.experimental.pallas import tpu as pltpu
from jax.experimental.pallas import tpu_sc as plsc

N_RAYS = 65536
N_PTS = 128
N_IMP = 64
ALPHA = 10.0
BETA = 0.1

NC = 2
NS = 16
NW = NC * NS
RAYS_PER_TILE = N_RAYS // NW
BLK = 16
NBLK = RAYS_PER_TILE // BLK


def _sc_body(d_hbm, sdf_hbm, u_hbm, out_hbm, d_v, sdf_v, cdf_v, out_v, u_v):
    wid = lax.axis_index("c") * NS + lax.axis_index("s")
    iota = lax.iota(jnp.int32, BLK)
    zeros_i = jnp.zeros((BLK,), jnp.int32)

    pltpu.sync_copy(u_hbm, u_v)

    def block_body(b, _):
        base = wid * RAYS_PER_TILE + b * BLK
        pltpu.sync_copy(d_hbm.at[pl.ds(base, BLK), :], d_v)
        pltpu.sync_copy(sdf_hbm.at[pl.ds(base, BLK), :], sdf_v)

        d0 = plsc.load_gather(d_v, [iota, zeros_i])

        def cdf_body(i, carry):
            run, dcur = carry
            plsc.store_scatter(cdf_v, [jnp.full((BLK,), i, jnp.int32), iota],
                               1.0 - jnp.exp(-run))
            dnext = plsc.load_gather(d_v, [iota, jnp.full((BLK,), i + 1, jnp.int32)])
            s = plsc.load_gather(sdf_v, [iota, jnp.full((BLK,), i, jnp.int32)])
            e = 0.5 * jnp.exp(-jnp.abs(s) * (1.0 / BETA))
            sigma = ALPHA * jnp.where(s >= 0, e, 1.0 - e)
            return run + sigma * (dnext - dcur), dnext

        lax.fori_loop(0, N_PTS - 1, cdf_body,
                      (jnp.zeros((BLK,), jnp.float32), d0))

        def q_body(qc, _):
            for j in range(16):
                q = qc * 16 + j
                qv = jnp.full((BLK,), q, jnp.int32)
                uq = plsc.load_gather(u_v, [qv])
                lo = jnp.zeros((BLK,), jnp.int32)
                hi = jnp.full((BLK,), N_PTS - 1, jnp.int32)
                for _step in range(7):
                    mid = lax.shift_right_logical(lo + hi, 1)
                    cm = plsc.load_gather(cdf_v, [mid, iota])
                    pred = cm < uq
                    lo = jnp.where(pred, mid + 1, lo)
                    hi = jnp.where(pred, hi, mid)
                below = jnp.maximum(lo - 1, 0)
                above = jnp.minimum(lo, N_PTS - 2)
                cdf0 = plsc.load_gather(cdf_v, [below, iota])
                cdf1 = plsc.load_gather(cdf_v, [above, iota])
                b0 = plsc.load_gather(d_v, [iota, below])
                b1 = plsc.load_gather(d_v, [iota, above])
                denom = cdf1 - cdf0
                denom = jnp.where(denom < 1e-5, 1.0, denom)
                t = (uq - cdf0) / denom
                plsc.store_scatter(out_v, [iota, qv], b0 + t * (b1 - b0))
            return 0

        lax.fori_loop(0, 4, q_body, 0)
        pltpu.sync_copy(out_v, out_hbm.at[pl.ds(base, BLK), :])
        return 0

    lax.fori_loop(0, NBLK, block_body, 0)


@jax.jit
def kernel(d_vals, sdf):
    u = jnp.linspace(0.0, 1.0, N_IMP, dtype=d_vals.dtype)
    mesh = plsc.VectorSubcoreMesh(core_axis_name="c", subcore_axis_name="s")
    fn = pl.kernel(
        _sc_body,
        out_type=jax.ShapeDtypeStruct((N_RAYS, N_IMP), jnp.float32),
        mesh=mesh,
        compiler_params=pltpu.CompilerParams(needs_layout_passes=False),
        scratch_types=[
            pltpu.VMEM((BLK, N_PTS), jnp.float32),
            pltpu.VMEM((BLK, N_PTS), jnp.float32),
            pltpu.VMEM((N_PTS, BLK), jnp.float32),
            pltpu.VMEM((BLK, N_IMP), jnp.float32),
            pltpu.VMEM((N_IMP,), jnp.float32),
        ],
    )
    return fn(d_vals, sdf, u)
